# baseline (speedup 1.0000x reference)
import functools

import jax
import jax.numpy as jnp
from jax import lax
from jax.experimental import pallas as pl
from jax.experimental.pallas import tpu as pltpu
from jax.experimental.pallas import tpu_sc as plsc

VOCAB = 100000
EMBED = 64
BATCH = 4096
CTX = 20

NC, NS = 2, 16
NW = NC * NS
BPW = BATCH // NW
HALF = BPW // 2
ROWS_PER_HALF = HALF * CTX
GATHERS_PER_HALF = ROWS_PER_HALF // 128

VB = 512
NVB = (VOCAB + VB - 1) // VB
NEG = -3.0e38


def _sc_body(idx_hbm, table_hbm, out_hbm, idx_v, rows_v, acc_v, sem):
    wid = lax.axis_index("s") * NC + lax.axis_index("c")
    pltpu.sync_copy(idx_hbm.at[pl.ds(wid * (BPW * CTX), BPW * CTX)], idx_v)
    for half in range(2):
        descs = []
        for g in range(GATHERS_PER_HALF):
            gg = half * GATHERS_PER_HALF + g
            descs.append(
                pltpu.async_copy(table_hbm.at[idx_v.at[pl.ds(gg * 128, 128)]],
                                 rows_v.at[pl.ds(g * 128, 128)], sem))
        for d_ in descs:
            d_.wait()

        def body(bb, carry):
            r0 = bb * CTX
            for d in range(EMBED // 16):
                acc = rows_v[r0, pl.ds(d * 16, 16)]
                for c in range(1, CTX):
                    acc = acc + rows_v[r0 + c, pl.ds(d * 16, 16)]
                acc_v[bb, pl.ds(d * 16, 16)] = acc * (1.0 / CTX)
            return carry

        lax.fori_loop(0, HALF, body, 0)
        pltpu.sync_copy(acc_v, out_hbm.at[pl.ds(wid * BPW + half * HALF, HALF)])


def _sc_avg(idx2d, table):
    mesh = plsc.VectorSubcoreMesh(core_axis_name="c", subcore_axis_name="s")
    return pl.kernel(
        _sc_body,
        out_type=jax.ShapeDtypeStruct((BATCH, EMBED), jnp.float32),
        mesh=mesh,
        scratch_types=[
            pltpu.VMEM((BPW * CTX,), jnp.int32),
            pltpu.VMEM((ROWS_PER_HALF, EMBED), jnp.float32),
            pltpu.VMEM((HALF, EMBED), jnp.float32),
            pltpu.SemaphoreType.DMA,
        ],
        compiler_params=pltpu.CompilerParams(use_tc_tiling_on_sc=False),
    )(idx2d, table)


def _pass_a_kernel(avg_ref, w_ref, b_ref, m_ref, r_ref):
    j = pl.program_id(0)
    l = jnp.dot(avg_ref[...].astype(jnp.bfloat16),
                w_ref[...].astype(jnp.bfloat16),
                preferred_element_type=jnp.float32)
    l = l + b_ref[...]
    col = lax.broadcasted_iota(jnp.int32, (1, VB), 1) + j * VB
    l = jnp.where(col < VOCAB, l, NEG)
    bm = jnp.max(l, axis=1, keepdims=True)

    @pl.when(j == 0)
    def _():
        m_ref[...] = bm
        r_ref[...] = jnp.sum(jnp.exp(l - bm), axis=1, keepdims=True)

    @pl.when(j > 0)
    def _():
        m_old = m_ref[...]
        m_new = jnp.maximum(m_old, bm)
        r_ref[...] = (r_ref[...] * jnp.exp(m_old - m_new)
                      + jnp.sum(jnp.exp(l - m_new), axis=1, keepdims=True))
        m_ref[...] = m_new

    @pl.when(j == NVB - 1)
    def _():
        r_ref[...] = 1.0 / r_ref[...]


def _pass_a(avg, w, b2):
    return pl.pallas_call(
        _pass_a_kernel,
        grid=(NVB,),
        in_specs=[
            pl.BlockSpec((BATCH, EMBED), lambda j: (0, 0)),
            pl.BlockSpec((EMBED, VB), lambda j: (0, j)),
            pl.BlockSpec((1, VB), lambda j: (0, j)),
        ],
        out_specs=[
            pl.BlockSpec((BATCH, 1), lambda j: (0, 0)),
            pl.BlockSpec((BATCH, 1), lambda j: (0, 0)),
        ],
        out_shape=[jax.ShapeDtypeStruct((BATCH, 1), jnp.float32)] * 2,
    )(avg, w, b2)


def _pass_b_kernel(avg_ref, w_ref, b_ref, m_ref, r_ref, out_ref):
    l = jnp.dot(avg_ref[...].astype(jnp.bfloat16),
                w_ref[...].astype(jnp.bfloat16),
                preferred_element_type=jnp.float32)
    l = l + b_ref[...]
    out_ref[...] = jnp.exp(l - m_ref[...]) * r_ref[...]


def _pass_b(avg, w, b2, m, r):
    return pl.pallas_call(
        _pass_b_kernel,
        grid=(NVB,),
        in_specs=[
            pl.BlockSpec((BATCH, EMBED), lambda j: (0, 0)),
            pl.BlockSpec((EMBED, VB), lambda j: (0, j)),
            pl.BlockSpec((1, VB), lambda j: (0, j)),
            pl.BlockSpec((BATCH, 1), lambda j: (0, 0)),
            pl.BlockSpec((BATCH, 1), lambda j: (0, 0)),
        ],
        out_specs=pl.BlockSpec((BATCH, VB), lambda j: (0, j)),
        out_shape=jax.ShapeDtypeStruct((BATCH, VOCAB), jnp.float32),
    )(avg, w, b2, m, r)


def kernel(inputs, table, W, b):
    idx_flat = inputs.astype(jnp.int32).reshape(BATCH * CTX)
    avg = _sc_avg(idx_flat, table)
    b2 = b.reshape(1, VOCAB)
    m, r = _pass_a(avg, W, b2)
    return _pass_b(avg, W, b2, m, r)
